# baseline (speedup 1.0000x reference)
import functools

import jax
import jax.numpy as jnp
from jax import lax
from jax.experimental import pallas as pl
from jax.experimental.pallas import tpu as pltpu
from jax.experimental.pallas import tpu_sc as plsc

NB = 4
N = 4096
D = 128
NID = 64
MARGIN_C = 0.3
EPS_C = 1e-08

NC = 2
NS = 16
NW = NC * NS
WROWS = NB * N // NW
CHUNK = 128
NCHUNK = WROWS // CHUNK
LANES = 16

@functools.lru_cache(maxsize=None)
def _build_sc_segment_sums():
    mesh = plsc.VectorSubcoreMesh(
        core_axis_name="c", subcore_axis_name="s", num_cores=NC, num_subcores=NS
    )
    return functools.partial(
        pl.kernel,
        out_type=jax.ShapeDtypeStruct((NW, NID * D), jnp.float32),
        mesh=mesh,
        compiler_params=pltpu.CompilerParams(needs_layout_passes=False),
        scratch_types=[
            pltpu.VMEM((WROWS, D), jnp.float32),
            pltpu.VMEM((WROWS,), jnp.int32),
            pltpu.VMEM((NID * D,), jnp.float32),
            pltpu.SemaphoreType.DMA,
            pltpu.SemaphoreType.DMA,
            pltpu.SemaphoreType.DMA,
            pltpu.SemaphoreType.DMA,
            pltpu.SemaphoreType.DMA,
        ],
    )(_sc_segment_sums_body)


def _sc_segment_sums_body(
    feats_hbm, tgt_hbm, out_hbm, fbuf, tbuf, acc_loc,
    sem_t, sem_a, sem_b, sem_c, sem_d
):
    cid = lax.axis_index("c")
    sid = lax.axis_index("s")
    wid = sid * NC + cid
    branch = wid // (NW // NB)
    group = lax.rem(wid, NW // NB)
    rbase = branch * N + group * WROWS
    tbase = group * WROWS

    cp_t = pltpu.async_copy(tgt_hbm.at[pl.ds(tbase, WROWS)], tbuf, sem_t)
    qsems = (sem_a, sem_b, sem_c, sem_d)
    qcps = [
        pltpu.async_copy(
            feats_hbm.at[pl.ds(rbase + q * CHUNK, CHUNK)],
            fbuf.at[pl.ds(q * CHUNK, CHUNK)],
            qsems[q],
        )
        for q in range(NCHUNK)
    ]

    zero_v = jnp.zeros((LANES,), jnp.float32)

    def _zero_body(r, carry):
        acc_loc[pl.ds(r * LANES, LANES)] = zero_v
        return carry

    lax.fori_loop(0, NID * D // LANES, _zero_body, 0)
    cp_t.wait()

    NV = D // LANES
    iota16 = lax.iota(jnp.int32, LANES)
    col_idx = [jnp.full((LANES,), v * LANES, jnp.int32) + iota16 for v in range(NV)]

    def _group_body(gg, carry):
        base = gg * LANES
        tvec = tbuf[pl.ds(base, LANES)]
        for r in range(LANES):
            tid_b = tvec.at[jnp.full((LANES,), r, jnp.int32)].get(
                mode="promise_in_bounds"
            )
            row_base = tid_b * D
            for v in range(NV):
                x = fbuf[base + r, pl.ds(v * LANES, LANES)]
                plsc.addupdate_scatter(acc_loc, [row_base + col_idx[v]], x)
        return carry

    gpc = CHUNK // LANES
    for q in range(NCHUNK):
        qcps[q].wait()
        lax.fori_loop(q * gpc, (q + 1) * gpc, _group_body, 0)

    pltpu.sync_copy(acc_loc, out_hbm.at[wid])


def _tc_loss_body(part_ref, tgt_ref, out_ref):
    tgt = tgt_ref[...]

    ids2 = lax.broadcasted_iota(jnp.int32, (NID, N), 0)
    onehot = (jnp.broadcast_to(tgt, (NID, N)) == ids2).astype(jnp.float32)
    counts = jnp.sum(onehot, axis=1, keepdims=True)
    present_f = jnp.minimum(counts, 1.0)
    denom = jnp.maximum(counts, 1.0)

    centers = []
    for b in range(NB):
        sb = part_ref[b, 0]
        for g in range(1, NW // NB):
            sb = sb + part_ref[b, g]
        centers.append(sb / denom)

    n_ids = jnp.sum(present_f)
    has_other_f = jnp.minimum(jnp.maximum(n_ids - present_f, 0.0), 1.0)
    contrib_f = present_f * has_other_f

    BIGP = jnp.float32(1e30)
    eye_f = (
        lax.broadcasted_iota(jnp.int32, (NID, NID), 0)
        == lax.broadcasted_iota(jnp.int32, (NID, NID), 1)
    ).astype(jnp.float32)
    pen = (eye_f + jnp.reshape(1.0 - present_f, (1, NID))) * BIGP

    hard = []
    for i in range(NB - 1):
        c = centers[i]
        sq = jnp.sum(c * c, axis=1, keepdims=True)
        gram = lax.dot_general(
            c, c, (((1,), (1,)), ((), ())),
            precision=lax.Precision.HIGHEST,
        )
        d2 = jnp.maximum(sq + jnp.reshape(sq, (1, NID)) - 2.0 * gram, 0.0)
        hard.append(jnp.sqrt(jnp.min(d2 + pen, axis=1, keepdims=True)))

    per_id = jnp.zeros((NID, 1), jnp.float32)
    for i in range(NB):
        for j in range(i + 1, NB):
            dij = centers[i] - centers[j] + EPS_C
            pos = jnp.sqrt(jnp.sum(dij * dij, axis=1, keepdims=True))
            per_id = per_id + jnp.maximum(MARGIN_C + pos - hard[i], 0.0)
    total = jnp.sum(per_id * contrib_f)

    pair_count = NB * (NB - 1) // 2
    valid_pairs = pair_count * jnp.where(n_ids > 1.0, n_ids, 0.0)
    safe_vp = jnp.maximum(valid_pairs, 1.0)
    loss = jnp.where(valid_pairs > 0.0, total / safe_vp, 0.0)
    out_ref[...] = jnp.reshape(loss, (1, 1))


_tc_loss = pl.pallas_call(
    _tc_loss_body,
    out_shape=jax.ShapeDtypeStruct((1, 1), jnp.float32),
)


def kernel(branch_feats, targets):
    t32 = targets.astype(jnp.int32)
    feats_flat = branch_feats.reshape(NB * N, D)
    partials = _build_sc_segment_sums()(feats_flat, t32)
    loss = _tc_loss(
        partials.reshape(NB, NW // NB, NID, D), t32.reshape(1, N)
    )
    return loss[0, 0]


# speedup vs baseline: 1.5157x; 1.5157x over previous
import functools

import jax
import jax.numpy as jnp
from jax import lax
from jax.experimental import pallas as pl
from jax.experimental.pallas import tpu as pltpu
from jax.experimental.pallas import tpu_sc as plsc

NB = 4
N = 4096
D = 128
NID = 64
MARGIN_C = 0.3
EPS_C = 1e-08

NC = 2
NS = 16
NW = NC * NS
WROWS = NB * N // NW
CHUNK = 128
NCHUNK = WROWS // CHUNK
LANES = 16

@functools.lru_cache(maxsize=None)
def _build_sc_segment_sums():
    mesh = plsc.VectorSubcoreMesh(
        core_axis_name="c", subcore_axis_name="s", num_cores=NC, num_subcores=NS
    )
    return functools.partial(
        pl.kernel,
        out_type=jax.ShapeDtypeStruct((NC, NB * NID, D), jnp.float32),
        mesh=mesh,
        scratch_types=[
            pltpu.VMEM((WROWS, D), jnp.float32),
            pltpu.VMEM((WROWS,), jnp.int32),
            pltpu.VMEM((NCHUNK, CHUNK), jnp.int32),
            pltpu.VMEM((NB * NID // NS, D), jnp.float32),
            pltpu.VMEM_SHARED((NB * NID, D), jnp.float32),
            pltpu.SemaphoreType.DMA,
            pltpu.SemaphoreType.DMA,
            pltpu.SemaphoreType.DMA,
            pltpu.SemaphoreType.DMA,
            pltpu.SemaphoreType.DMA,
            pltpu.SemaphoreType.DMA,
        ],
    )(_sc_segment_sums_body)


def _sc_segment_sums_body(
    feats_hbm, tgt_hbm, out_hbm, fbuf, tbuf, ibuf, zbuf, acc,
    sem_t, sem_a, sem_b, sem_c, sem_d, sem_s
):
    cid = lax.axis_index("c")
    sid = lax.axis_index("s")
    wid = sid * NC + cid
    branch = wid // (NW // NB)
    group = lax.rem(wid, NW // NB)
    rbase = branch * N + group * WROWS
    tbase = group * WROWS

    cp_t = pltpu.async_copy(tgt_hbm.at[pl.ds(tbase, WROWS)], tbuf, sem_t)
    qsems = (sem_a, sem_b, sem_c, sem_d)
    qcps = [
        pltpu.async_copy(
            feats_hbm.at[pl.ds(rbase + q * CHUNK, CHUNK)],
            fbuf.at[pl.ds(q * CHUNK, CHUNK)],
            qsems[q],
        )
        for q in range(NCHUNK)
    ]

    zrows = NB * NID // NS
    zero_v = jnp.zeros((LANES,), jnp.float32)
    for r in range(zrows):
        for v in range(D // LANES):
            zbuf[r, pl.ds(v * LANES, LANES)] = zero_v
    pltpu.sync_copy(zbuf, acc.at[pl.ds(sid * zrows, zrows)])

    cp_t.wait()
    off = jnp.full((LANES,), branch * NID, jnp.int32)
    for q in range(NCHUNK):
        for v in range(CHUNK // LANES):
            ibuf[q, pl.ds(v * LANES, LANES)] = tbuf[pl.ds(q * CHUNK + v * LANES, LANES)] + off
    plsc.subcore_barrier()

    scps = []
    for q in range(NCHUNK):
        qcps[q].wait()
        scps.append(
            pltpu.async_copy(
                fbuf.at[pl.ds(q * CHUNK, CHUNK)], acc.at[ibuf.at[q]], sem_s, add=True
            )
        )
    for cp in scps:
        cp.wait()
    plsc.subcore_barrier()

    @pl.when(sid == 0)
    def _():
        pltpu.sync_copy(acc, out_hbm.at[cid])


def _tc_loss_body(part_ref, tgt_ref, out_ref):
    sums = part_ref[0] + part_ref[1]
    tgt = tgt_ref[...]

    ids2 = lax.broadcasted_iota(jnp.int32, (NID, N), 0)
    onehot = (jnp.broadcast_to(tgt, (NID, N)) == ids2).astype(jnp.float32)
    counts = jnp.sum(onehot, axis=1, keepdims=True)
    present_f = jnp.minimum(counts, 1.0)
    denom = jnp.maximum(counts, 1.0)

    centers = [sums[b * NID:(b + 1) * NID, :] / denom for b in range(NB)]

    n_ids = jnp.sum(present_f)
    has_other_f = jnp.minimum(jnp.maximum(n_ids - present_f, 0.0), 1.0)
    contrib_f = present_f * has_other_f

    BIGP = jnp.float32(1e30)
    eye_f = (
        lax.broadcasted_iota(jnp.int32, (NID, NID), 0)
        == lax.broadcasted_iota(jnp.int32, (NID, NID), 1)
    ).astype(jnp.float32)
    pen = (eye_f + jnp.reshape(1.0 - present_f, (1, NID))) * BIGP

    hard = []
    for i in range(NB - 1):
        c = centers[i]
        sq = jnp.sum(c * c, axis=1, keepdims=True)
        gram = lax.dot_general(
            c, c, (((1,), (1,)), ((), ())),
            precision=lax.Precision.HIGHEST,
        )
        d2 = jnp.maximum(sq + jnp.reshape(sq, (1, NID)) - 2.0 * gram, 0.0)
        hard.append(jnp.sqrt(jnp.min(d2 + pen, axis=1, keepdims=True)))

    per_id = jnp.zeros((NID, 1), jnp.float32)
    for i in range(NB):
        for j in range(i + 1, NB):
            dij = centers[i] - centers[j] + EPS_C
            pos = jnp.sqrt(jnp.sum(dij * dij, axis=1, keepdims=True))
            per_id = per_id + jnp.maximum(MARGIN_C + pos - hard[i], 0.0)
    total = jnp.sum(per_id * contrib_f)

    pair_count = NB * (NB - 1) // 2
    valid_pairs = pair_count * jnp.where(n_ids > 1.0, n_ids, 0.0)
    safe_vp = jnp.maximum(valid_pairs, 1.0)
    loss = jnp.where(valid_pairs > 0.0, total / safe_vp, 0.0)
    out_ref[...] = jnp.reshape(loss, (1, 1))


_tc_loss = pl.pallas_call(
    _tc_loss_body,
    out_shape=jax.ShapeDtypeStruct((1, 1), jnp.float32),
)


def kernel(branch_feats, targets):
    t32 = targets.astype(jnp.int32)
    feats_flat = branch_feats.reshape(NB * N, D)
    partials = _build_sc_segment_sums()(feats_flat, t32)
    loss = _tc_loss(partials, t32.reshape(1, N))
    return loss[0, 0]
